# baseline (speedup 1.0000x reference)
import functools

import jax
import jax.numpy as jnp
from jax import lax
from jax.experimental import pallas as pl
from jax.experimental.pallas import tpu as pltpu
from jax.experimental.pallas import tpu_sc as plsc

B = 16
VOCAB = 1000
E = 32
SPATIAL = 32 * 32 * 32

NC, NS, L = 2, 16, 16
NW = NC * NS
CHUNK = SPATIAL // NW
G = CHUNK // L
PITCH = CHUNK + 1


def _sc_embed(ids, table_flat):
    mesh = plsc.VectorSubcoreMesh(core_axis_name="c", subcore_axis_name="s")

    @functools.partial(
        pl.kernel,
        mesh=mesh,
        out_type=jax.ShapeDtypeStruct((B, E, SPATIAL), jnp.float32),
        compiler_params=pltpu.CompilerParams(needs_layout_passes=False),
        scratch_types=[
            pltpu.VMEM((VOCAB * E,), jnp.float32),
            pltpu.VMEM((2, CHUNK), jnp.int32),
            pltpu.VMEM((2, E, PITCH), jnp.float32),
            pltpu.SemaphoreType.DMA,
            pltpu.SemaphoreType.DMA,
            pltpu.SemaphoreType.DMA,
            pltpu.SemaphoreType.DMA,
        ],
    )
    def k(ids_hbm, tbl_hbm, out_hbm, tbl_v, ids_v, out_v,
          sem_i0, sem_i1, sem_o0, sem_o1):
        sem_i = (sem_i0, sem_i1)
        sem_o = (sem_o0, sem_o1)
        wid = lax.axis_index("s") * NC + lax.axis_index("c")
        base_j = pl.multiple_of(wid * CHUNK, 8)
        pltpu.sync_copy(tbl_hbm, tbl_v)

        iota = lax.broadcasted_iota(jnp.int32, (L,), 0)
        half = tuple(iota + h * L for h in range(2))

        def start_ids(b, u):
            pltpu.async_copy(
                ids_hbm.at[b, pl.ds(base_j, CHUNK)], ids_v.at[u], sem_i[u])

        def start_out(b, u):
            pltpu.async_copy(
                out_v.at[u, :, pl.ds(0, CHUNK)],
                out_hbm.at[b, :, pl.ds(base_j, CHUNK)], sem_o[u])

        def wait_ids(u):
            pltpu.make_async_copy(
                ids_hbm.at[0, pl.ds(base_j, CHUNK)], ids_v.at[u],
                sem_i[u]).wait()

        def wait_out(u):
            pltpu.make_async_copy(
                out_v.at[u, :, pl.ds(0, CHUNK)],
                out_hbm.at[0, :, pl.ds(base_j, CHUNK)], sem_o[u]).wait()

        start_ids(0, 0)
        start_ids(1, 1)

        @pl.loop(0, B, step=2)
        def _(bb):
            for u in range(2):
                b = bb + u
                wait_ids(u)

                @pl.when(bb > 0)
                def _():
                    wait_out(u)

                @plsc.parallel_loop(0, G, unroll=1)
                def _(g):
                    idx = ids_v[u, pl.ds(g * L, L)]
                    idx = jnp.minimum(jnp.maximum(idx, 0), VOCAB - 1)
                    base = idx * E
                    j0 = g * L
                    for l in range(L):
                        bid = jnp.take_along_axis(
                            base, jnp.full((L,), l, jnp.int32), axis=0)
                        j_vec = jnp.broadcast_to(j0 + l, (L,))
                        for h in range(2):
                            val = plsc.load_gather(tbl_v, [bid + half[h]])
                            plsc.store_scatter(
                                out_v.at[u], [half[h], j_vec], val)

                start_out(b, u)

                @pl.when(bb < B - 2)
                def _():
                    start_ids(b + 2, u)

        wait_out(0)
        wait_out(1)

    return k(ids, table_flat)


def kernel(parent_blocks, table):
    ids = parent_blocks.astype(jnp.int32).reshape(B, SPATIAL)
    out = _sc_embed(ids, table.reshape(-1))
    return out.reshape(B, E, 32, 32, 32)


# speedup vs baseline: 1.8700x; 1.8700x over previous
import functools

import jax
import jax.numpy as jnp
from jax import lax
from jax.experimental import pallas as pl
from jax.experimental.pallas import tpu as pltpu
from jax.experimental.pallas import tpu_sc as plsc

B = 16
VOCAB = 1000
E = 32
SPATIAL = 32 * 32 * 32

NC, NS, L = 2, 16, 16
NW = NC * NS
CHUNK = SPATIAL // NW
G = CHUNK // L


def _sc_embed(ids, table_flat):
    mesh = plsc.VectorSubcoreMesh(core_axis_name="c", subcore_axis_name="s")

    @functools.partial(
        pl.kernel,
        mesh=mesh,
        out_type=jax.ShapeDtypeStruct((B, E, SPATIAL), jnp.float32),
        compiler_params=pltpu.CompilerParams(needs_layout_passes=False),
        scratch_types=[
            pltpu.VMEM((VOCAB * E,), jnp.float32),
            pltpu.VMEM((2, CHUNK), jnp.int32),
            pltpu.VMEM((2, E, CHUNK), jnp.float32),
            pltpu.SemaphoreType.DMA,
            pltpu.SemaphoreType.DMA,
            pltpu.SemaphoreType.DMA,
            pltpu.SemaphoreType.DMA,
        ],
    )
    def k(ids_hbm, tbl_hbm, out_hbm, tbl_v, ids_v, out_v,
          sem_i0, sem_i1, sem_o0, sem_o1):
        sem_i = (sem_i0, sem_i1)
        sem_o = (sem_o0, sem_o1)
        wid = lax.axis_index("s") * NC + lax.axis_index("c")
        base_j = pl.multiple_of(wid * CHUNK, 8)
        pltpu.sync_copy(tbl_hbm, tbl_v)

        def start_ids(b, u):
            pltpu.async_copy(
                ids_hbm.at[b, pl.ds(base_j, CHUNK)], ids_v.at[u], sem_i[u])

        def start_out(b, u):
            pltpu.async_copy(
                out_v.at[u], out_hbm.at[b, :, pl.ds(base_j, CHUNK)], sem_o[u])

        def wait_ids(u):
            pltpu.make_async_copy(
                ids_hbm.at[0, pl.ds(base_j, CHUNK)], ids_v.at[u],
                sem_i[u]).wait()

        def wait_out(u):
            pltpu.make_async_copy(
                out_v.at[u], out_hbm.at[0, :, pl.ds(base_j, CHUNK)],
                sem_o[u]).wait()

        start_ids(0, 0)
        start_ids(1, 1)

        @pl.loop(0, B, step=2)
        def _(bb):
            for u in range(2):
                b = bb + u
                wait_ids(u)

                @pl.when(bb > 0)
                def _():
                    wait_out(u)

                @plsc.parallel_loop(0, G, unroll=2)
                def _(g):
                    idx = ids_v[u, pl.ds(g * L, L)]
                    idx = jnp.minimum(jnp.maximum(idx, 0), VOCAB - 1)
                    for e in range(E):
                        out_v[u, e, pl.ds(g * L, L)] = plsc.load_gather(
                            tbl_v, [idx + e * VOCAB]
                        )

                start_out(b, u)

                @pl.when(bb < B - 2)
                def _():
                    start_ids(b + 2, u)

        wait_out(0)
        wait_out(1)

    return k(ids, table_flat)


def kernel(parent_blocks, table):
    ids = parent_blocks.astype(jnp.int32).reshape(B, SPATIAL)
    out = _sc_embed(ids, table.T.reshape(-1))
    return out.reshape(B, E, 32, 32, 32)


# speedup vs baseline: 3.3038x; 1.7667x over previous
import functools

import jax
import jax.numpy as jnp
from jax import lax
from jax.experimental import pallas as pl
from jax.experimental.pallas import tpu as pltpu
from jax.experimental.pallas import tpu_sc as plsc

B = 16
VOCAB = 1000
E = 32
DIM = 32
SPATIAL = DIM * DIM * DIM
EB = 8
NB = E // EB

NC, NS, L = 2, 16, 16
NW = NC * NS
CHUNK = SPATIAL // NW
G = CHUNK // L


def _sc_embed(ids, table_flat):
    mesh = plsc.VectorSubcoreMesh(core_axis_name="c", subcore_axis_name="s")

    @functools.partial(
        pl.kernel,
        mesh=mesh,
        out_type=jax.ShapeDtypeStruct((B, E, DIM, DIM, DIM), jnp.float32),
        compiler_params=pltpu.CompilerParams(needs_layout_passes=False),
        scratch_types=[
            pltpu.VMEM((VOCAB * E,), jnp.float32),
            pltpu.VMEM((2, CHUNK), jnp.int32),
            pltpu.VMEM((2, EB, DIM, DIM), jnp.float32),
            pltpu.SemaphoreType.DMA,
            pltpu.SemaphoreType.DMA,
            pltpu.SemaphoreType.DMA,
            pltpu.SemaphoreType.DMA,
        ],
    )
    def k(ids_hbm, tbl_hbm, out_hbm, tbl_v, ids_v, out_v,
          sem_i0, sem_i1, sem_o0, sem_o1):
        sem_i = (sem_i0, sem_i1)
        sem_o = (sem_o0, sem_o1)
        wid = lax.axis_index("s") * NC + lax.axis_index("c")
        base_j = pl.multiple_of(wid * CHUNK, 8)
        pltpu.sync_copy(tbl_hbm, tbl_v)

        def start_ids(b, u):
            pltpu.async_copy(
                ids_hbm.at[b, pl.ds(base_j, CHUNK)], ids_v.at[u], sem_i[u])

        def start_out(b, kb, v):
            pltpu.async_copy(
                out_v.at[v], out_hbm.at[b, pl.ds(kb * EB, EB), wid],
                sem_o[v])

        def wait_ids(u):
            pltpu.make_async_copy(
                ids_hbm.at[0, pl.ds(base_j, CHUNK)], ids_v.at[u],
                sem_i[u]).wait()

        def wait_out(v):
            pltpu.make_async_copy(
                out_v.at[v], out_hbm.at[0, pl.ds(0, EB), 0], sem_o[v]).wait()

        start_ids(0, 0)
        start_ids(1, 1)

        @pl.loop(0, B, step=2)
        def _(bb):
            for u in range(2):
                b = bb + u
                wait_ids(u)
                for kb in range(NB):
                    v = kb % 2

                    @pl.when(jnp.logical_or(bb > 0, (u * NB + kb) >= 2))
                    def _():
                        wait_out(v)

                    @plsc.parallel_loop(0, G, unroll=2)
                    def _(g):
                        idx = ids_v[u, pl.ds(g * L, L)]
                        idx = jnp.minimum(jnp.maximum(idx, 0), VOCAB - 1)
                        y = g // 2
                        z0 = (g % 2) * L
                        for el in range(EB):
                            out_v[v, el, y, pl.ds(z0, L)] = plsc.load_gather(
                                tbl_v, [idx + (kb * EB + el) * VOCAB]
                            )

                    start_out(b, kb, v)

                @pl.when(bb < B - 2)
                def _():
                    start_ids(b + 2, u)

        wait_out(0)
        wait_out(1)

    return k(ids, table_flat)


def kernel(parent_blocks, table):
    ids = parent_blocks.astype(jnp.int32).reshape(B, SPATIAL)
    return _sc_embed(ids, table.T.reshape(-1))


# speedup vs baseline: 3.3405x; 1.0111x over previous
import functools

import jax
import jax.numpy as jnp
from jax import lax
from jax.experimental import pallas as pl
from jax.experimental.pallas import tpu as pltpu
from jax.experimental.pallas import tpu_sc as plsc

B = 16
VOCAB = 1000
E = 32
DIM = 32
SPATIAL = DIM * DIM * DIM
EB = 8
NB = E // EB

NC, NS, L = 2, 16, 16
NW = NC * NS
CHUNK = SPATIAL // NW
G = CHUNK // L


def _sc_embed(ids, table_flat):
    mesh = plsc.VectorSubcoreMesh(core_axis_name="c", subcore_axis_name="s")

    @functools.partial(
        pl.kernel,
        mesh=mesh,
        out_type=jax.ShapeDtypeStruct((B, E, DIM, DIM, DIM), jnp.float32),
        compiler_params=pltpu.CompilerParams(needs_layout_passes=False),
        scratch_types=[
            pltpu.VMEM((VOCAB * E,), jnp.float32),
            pltpu.VMEM((2, DIM, DIM), jnp.int32),
            pltpu.VMEM((2, EB, DIM, DIM), jnp.float32),
            pltpu.SemaphoreType.DMA,
            pltpu.SemaphoreType.DMA,
            pltpu.SemaphoreType.DMA,
            pltpu.SemaphoreType.DMA,
        ],
    )
    def k(ids_hbm, tbl_hbm, out_hbm, tbl_v, ids_v, out_v,
          sem_i0, sem_i1, sem_o0, sem_o1):
        sem_i = (sem_i0, sem_i1)
        sem_o = (sem_o0, sem_o1)
        wid = lax.axis_index("s") * NC + lax.axis_index("c")
        pltpu.sync_copy(tbl_hbm, tbl_v)

        def start_ids(b, u):
            pltpu.async_copy(ids_hbm.at[b, wid], ids_v.at[u], sem_i[u])

        def start_out(b, kb, v):
            pltpu.async_copy(
                out_v.at[v], out_hbm.at[b, pl.ds(kb * EB, EB), wid],
                sem_o[v])

        def wait_ids(u):
            pltpu.make_async_copy(
                ids_hbm.at[0, 0], ids_v.at[u], sem_i[u]).wait()

        def wait_out(v):
            pltpu.make_async_copy(
                out_v.at[v], out_hbm.at[0, pl.ds(0, EB), 0], sem_o[v]).wait()

        start_ids(0, 0)
        start_ids(1, 1)

        @pl.loop(0, B, step=2)
        def _(bb):
            for u in range(2):
                b = bb + u
                wait_ids(u)
                for kb in range(NB):
                    v = kb % 2

                    @pl.when(jnp.logical_or(bb > 0, (u * NB + kb) >= 2))
                    def _():
                        wait_out(v)

                    @plsc.parallel_loop(0, G, unroll=2)
                    def _(g):
                        y = g // 2
                        z0 = (g % 2) * L
                        idx = ids_v[u, y, pl.ds(z0, L)]
                        idx = jnp.minimum(jnp.maximum(idx, 0), VOCAB - 1)
                        for el in range(EB):
                            out_v[v, el, y, pl.ds(z0, L)] = plsc.load_gather(
                                tbl_v, [idx + (kb * EB + el) * VOCAB]
                            )

                    start_out(b, kb, v)

                @pl.when(bb < B - 2)
                def _():
                    start_ids(b + 2, u)

        wait_out(0)
        wait_out(1)

    return k(ids, table_flat)


def kernel(parent_blocks, table):
    ids = parent_blocks.astype(jnp.int32)
    return _sc_embed(ids, table.T.reshape(-1))
